# baseline (device time: 79460 ns/iter reference)
import jax
import jax.numpy as jnp
from jax import lax
from jax.experimental import pallas as pl
from jax.experimental.pallas import tpu as pltpu

N_DEV = 4
SQ = 256
D = 1024
SKV = 4096
DH = 128
HQ_SHARD = 8
KV_SHARD = 2
SCALE = 0.08838834764831843


def kernel(x, Wq, Wo, K_ext, V_ext):
    my_pos = lax.axis_index("i")
    K_my = lax.dynamic_slice_in_dim(K_ext, 2 * my_pos, KV_SHARD, axis=2)
    V_my = lax.dynamic_slice_in_dim(V_ext, 2 * my_pos, KV_SHARD, axis=2)
    x2 = x.reshape(SQ, D)
    K2 = K_my.reshape(SKV, KV_SHARD * DH)
    V2 = V_my.reshape(SKV, KV_SHARD * DH)

    def body(x_ref, wq_ref, wo_ref, k_ref, v_ref, out_ref,
             attn_ref, comm_ref, send_sems, recv_sems):
        my = lax.axis_index("i")
        left = lax.rem(my + N_DEV - 1, N_DEV)
        right = lax.rem(my + 1, N_DEV)

        barrier = pltpu.get_barrier_semaphore()
        for nbr in (left, right):
            pl.semaphore_signal(barrier, inc=1, device_id=(nbr,),
                                device_id_type=pl.DeviceIdType.MESH)
        pl.semaphore_wait(barrier, 2)

        q = jnp.dot(x_ref[:, :], wq_ref[:, :],
                    preferred_element_type=jnp.float32)

        for h in range(HQ_SHARD):
            g = h // 4
            q_h = q[:, h * DH:(h + 1) * DH]
            k_h = k_ref[:, g * DH:(g + 1) * DH]
            v_h = v_ref[:, g * DH:(g + 1) * DH]
            s = lax.dot_general(q_h, k_h, (((1,), (1,)), ((), ())),
                                preferred_element_type=jnp.float32) * SCALE
            m = jnp.max(s, axis=1, keepdims=True)
            p = jnp.exp(s - m)
            l = jnp.sum(p, axis=1, keepdims=True)
            o = jnp.dot(p, v_h, preferred_element_type=jnp.float32) / l
            attn_ref[:, h * DH:(h + 1) * DH] = o

        comm_ref[0] = jnp.dot(attn_ref[:, :], wo_ref[:, :],
                              preferred_element_type=jnp.float32)

        for hop in range(N_DEV - 1):
            rdma = pltpu.make_async_remote_copy(
                src_ref=comm_ref.at[hop],
                dst_ref=comm_ref.at[hop + 1],
                send_sem=send_sems.at[hop],
                recv_sem=recv_sems.at[hop + 1],
                device_id=(right,),
                device_id_type=pl.DeviceIdType.MESH,
            )
            rdma.start()
            rdma.wait()

        out_ref[:, :] = ((comm_ref[0] + comm_ref[1])
                         + (comm_ref[2] + comm_ref[3]))

    out = pl.pallas_call(
        body,
        out_shape=jax.ShapeDtypeStruct((SQ, D), jnp.float32),
        in_specs=[pl.BlockSpec(memory_space=pltpu.VMEM)] * 5,
        out_specs=pl.BlockSpec(memory_space=pltpu.VMEM),
        scratch_shapes=[
            pltpu.VMEM((SQ, D), jnp.float32),
            pltpu.VMEM((N_DEV, SQ, D), jnp.float32),
            pltpu.SemaphoreType.DMA((N_DEV,)),
            pltpu.SemaphoreType.DMA((N_DEV,)),
        ],
        compiler_params=pltpu.CompilerParams(collective_id=0),
    )(x2, Wq, Wo, K2, V2)
    return out.reshape(1, SQ, D)


# device time: 74833 ns/iter; 1.0618x vs baseline; 1.0618x over previous
import jax
import jax.numpy as jnp
from jax import lax
from jax.experimental import pallas as pl
from jax.experimental.pallas import tpu as pltpu

N_DEV = 4
SQ = 256
D = 1024
SKV = 4096
DH = 128
HQ_SHARD = 8
KV_SHARD = 2
QC = D // 4
SCALE = 0.08838834764831843


def kernel(x, Wq, Wo, K_ext, V_ext):
    my_pos = lax.axis_index("i").astype(jnp.int32)
    x2 = x.reshape(SQ, D)
    K2 = K_ext.reshape(SKV, 8 * DH)
    V2 = V_ext.reshape(SKV, 8 * DH)

    def body(pos_ref, x_ref, wq_ref, wo_ref, k_ref, v_ref, out_ref,
             attn_ref, acc_ref, recv1_ref, recv2_ref, send_sems, recv_sems):
        my = pos_ref[0]
        b0 = lax.rem(my, 2)
        b1 = my // 2
        p1 = jnp.bitwise_xor(my, 1)
        p2 = jnp.bitwise_xor(my, 2)
        half_lo = 2 * b0
        oth_lo = 2 * (1 - b0)
        q_my = half_lo + b1
        q_oth = half_lo + (1 - b1)

        barrier = pltpu.get_barrier_semaphore()
        for nbr in (p1, p2):
            pl.semaphore_signal(barrier, inc=1, device_id=(nbr,),
                                device_id_type=pl.DeviceIdType.MESH)
        pl.semaphore_wait(barrier, 2)

        q = jnp.dot(x_ref[:, :], wq_ref[:, :],
                    preferred_element_type=jnp.float32)

        for h in range(HQ_SHARD):
            g = h // 4
            q_h = q[:, h * DH:(h + 1) * DH]
            k_h = k_ref[:, g * DH:(g + 1) * DH]
            v_h = v_ref[:, g * DH:(g + 1) * DH]
            s = lax.dot_general(q_h, k_h, (((1,), (1,)), ((), ())),
                                preferred_element_type=jnp.float32) * SCALE
            m = jnp.max(s, axis=1, keepdims=True)
            p = jnp.exp(s - m)
            l = jnp.sum(p, axis=1, keepdims=True)
            o = jnp.dot(p, v_h, preferred_element_type=jnp.float32) / l
            attn_ref[:, h * DH:(h + 1) * DH] = o

        partial = jnp.dot(attn_ref[:, :], wo_ref[:, :],
                          preferred_element_type=jnp.float32)
        for qq in range(N_DEV):
            acc_ref[qq] = partial[:, qq * QC:(qq + 1) * QC]

        rs1 = pltpu.make_async_remote_copy(
            src_ref=acc_ref.at[pl.ds(oth_lo, 2)],
            dst_ref=recv1_ref,
            send_sem=send_sems.at[0],
            recv_sem=recv_sems.at[0],
            device_id=(p1,),
            device_id_type=pl.DeviceIdType.MESH,
        )
        rs1.start()
        rs1.wait()
        acc_ref[pl.ds(half_lo, 2)] = acc_ref[pl.ds(half_lo, 2)] + recv1_ref[:, :, :]

        rs2 = pltpu.make_async_remote_copy(
            src_ref=acc_ref.at[pl.ds(q_oth, 1)],
            dst_ref=recv2_ref,
            send_sem=send_sems.at[1],
            recv_sem=recv_sems.at[1],
            device_id=(p2,),
            device_id_type=pl.DeviceIdType.MESH,
        )
        rs2.start()
        rs2.wait()
        acc_ref[pl.ds(q_my, 1)] = acc_ref[pl.ds(q_my, 1)] + recv2_ref[:, :, :]

        ag3 = pltpu.make_async_remote_copy(
            src_ref=acc_ref.at[pl.ds(q_my, 1)],
            dst_ref=acc_ref.at[pl.ds(q_my, 1)],
            send_sem=send_sems.at[2],
            recv_sem=recv_sems.at[2],
            device_id=(p2,),
            device_id_type=pl.DeviceIdType.MESH,
        )
        ag3.start()
        ag3.wait()

        ag4 = pltpu.make_async_remote_copy(
            src_ref=acc_ref.at[pl.ds(half_lo, 2)],
            dst_ref=acc_ref.at[pl.ds(half_lo, 2)],
            send_sem=send_sems.at[3],
            recv_sem=recv_sems.at[3],
            device_id=(p1,),
            device_id_type=pl.DeviceIdType.MESH,
        )
        ag4.start()
        ag4.wait()

        for qq in range(N_DEV):
            out_ref[:, qq * QC:(qq + 1) * QC] = acc_ref[qq]

    grid_spec = pltpu.PrefetchScalarGridSpec(
        num_scalar_prefetch=1,
        grid=(1,),
        in_specs=[
            pl.BlockSpec((SQ, D), lambda i, m: (0, 0)),
            pl.BlockSpec((D, D), lambda i, m: (0, 0)),
            pl.BlockSpec((D, D), lambda i, m: (0, 0)),
            pl.BlockSpec((SKV, KV_SHARD * DH), lambda i, m: (0, m[0])),
            pl.BlockSpec((SKV, KV_SHARD * DH), lambda i, m: (0, m[0])),
        ],
        out_specs=pl.BlockSpec((SQ, D), lambda i, m: (0, 0)),
        scratch_shapes=[
            pltpu.VMEM((SQ, D), jnp.float32),
            pltpu.VMEM((N_DEV, SQ, QC), jnp.float32),
            pltpu.VMEM((2, SQ, QC), jnp.float32),
            pltpu.VMEM((1, SQ, QC), jnp.float32),
            pltpu.SemaphoreType.DMA((N_DEV,)),
            pltpu.SemaphoreType.DMA((N_DEV,)),
        ],
    )
    out = pl.pallas_call(
        body,
        grid_spec=grid_spec,
        out_shape=jax.ShapeDtypeStruct((SQ, D), jnp.float32),
        compiler_params=pltpu.CompilerParams(collective_id=0),
    )(my_pos.reshape(1), x2, Wq, Wo, K2, V2)
    return out.reshape(1, SQ, D)


# device time: 50593 ns/iter; 1.5706x vs baseline; 1.4791x over previous
import jax
import jax.numpy as jnp
from jax import lax
from jax.experimental import pallas as pl
from jax.experimental.pallas import tpu as pltpu

N_DEV = 4
SQ = 256
D = 1024
SKV = 4096
DH = 128
HQ_SHARD = 8
KV_SHARD = 2
QC = D // 4
SCALE = 0.08838834764831843


def kernel(x, Wq, Wo, K_ext, V_ext):
    my_pos = lax.axis_index("i").astype(jnp.int32)

    def body(pos_ref, x_ref, wq_ref, wo_ref, k_any, v_any, out_ref,
             kv_ref, vv_ref, attn_ref, acc_ref, recv1_ref, recv2_ref,
             kv_sems, send_sems, recv_sems):
        my = pos_ref[0]
        b0 = lax.rem(my, 2)
        b1 = my // 2
        p1 = jnp.bitwise_xor(my, 1)
        p2 = jnp.bitwise_xor(my, 2)
        half_lo = 2 * b0
        oth_lo = 2 * (1 - b0)
        q_my = half_lo + b1
        q_oth = half_lo + (1 - b1)

        kv_copies = []
        for g in range(KV_SHARD):
            hd = 2 * my + g
            for j, (src, dst) in enumerate(((k_any, kv_ref), (v_any, vv_ref))):
                cp = pltpu.make_async_copy(
                    src.at[0, :, hd, :], dst.at[g], kv_sems.at[2 * g + j])
                cp.start()
                kv_copies.append(cp)

        barrier = pltpu.get_barrier_semaphore()
        for nbr in (p1, p2):
            pl.semaphore_signal(barrier, inc=1, device_id=(nbr,),
                                device_id_type=pl.DeviceIdType.MESH)
        pl.semaphore_wait(barrier, 2)

        q = jnp.dot(x_ref[0], wq_ref[:, :],
                    preferred_element_type=jnp.float32)

        for cp in kv_copies:
            cp.wait()

        for h in range(HQ_SHARD):
            g = h // 4
            q_h = q[:, h * DH:(h + 1) * DH]
            k_h = kv_ref[g]
            v_h = vv_ref[g]
            s = lax.dot_general(q_h, k_h, (((1,), (1,)), ((), ())),
                                preferred_element_type=jnp.float32) * SCALE
            m = jnp.max(s, axis=1, keepdims=True)
            p = jnp.exp(s - m)
            l = jnp.sum(p, axis=1, keepdims=True)
            o = jnp.dot(p, v_h, preferred_element_type=jnp.float32) / l
            attn_ref[:, h * DH:(h + 1) * DH] = o

        partial = jnp.dot(attn_ref[:, :], wo_ref[:, :],
                          preferred_element_type=jnp.float32)
        for qq in range(N_DEV):
            acc_ref[qq] = partial[:, qq * QC:(qq + 1) * QC]

        rs1 = pltpu.make_async_remote_copy(
            src_ref=acc_ref.at[pl.ds(oth_lo, 2)],
            dst_ref=recv1_ref,
            send_sem=send_sems.at[0],
            recv_sem=recv_sems.at[0],
            device_id=(p1,),
            device_id_type=pl.DeviceIdType.MESH,
        )
        rs1.start()
        rs1.wait()
        acc_ref[pl.ds(half_lo, 2)] = acc_ref[pl.ds(half_lo, 2)] + recv1_ref[:, :, :]

        rs2 = pltpu.make_async_remote_copy(
            src_ref=acc_ref.at[pl.ds(q_oth, 1)],
            dst_ref=recv2_ref,
            send_sem=send_sems.at[1],
            recv_sem=recv_sems.at[1],
            device_id=(p2,),
            device_id_type=pl.DeviceIdType.MESH,
        )
        rs2.start()
        rs2.wait()
        acc_ref[pl.ds(q_my, 1)] = acc_ref[pl.ds(q_my, 1)] + recv2_ref[:, :, :]

        ag3 = pltpu.make_async_remote_copy(
            src_ref=acc_ref.at[pl.ds(q_my, 1)],
            dst_ref=acc_ref.at[pl.ds(q_my, 1)],
            send_sem=send_sems.at[2],
            recv_sem=recv_sems.at[2],
            device_id=(p2,),
            device_id_type=pl.DeviceIdType.MESH,
        )
        ag3.start()
        ag3.wait()

        ag4 = pltpu.make_async_remote_copy(
            src_ref=acc_ref.at[pl.ds(half_lo, 2)],
            dst_ref=acc_ref.at[pl.ds(half_lo, 2)],
            send_sem=send_sems.at[3],
            recv_sem=recv_sems.at[3],
            device_id=(p1,),
            device_id_type=pl.DeviceIdType.MESH,
        )
        ag4.start()
        ag4.wait()

        for qq in range(N_DEV):
            out_ref[0, :, qq * QC:(qq + 1) * QC] = acc_ref[qq]

    grid_spec = pltpu.PrefetchScalarGridSpec(
        num_scalar_prefetch=1,
        grid=(1,),
        in_specs=[
            pl.BlockSpec((1, SQ, D), lambda i, m: (0, 0, 0)),
            pl.BlockSpec((D, D), lambda i, m: (0, 0)),
            pl.BlockSpec((D, D), lambda i, m: (0, 0)),
            pl.BlockSpec(memory_space=pl.ANY),
            pl.BlockSpec(memory_space=pl.ANY),
        ],
        out_specs=pl.BlockSpec((1, SQ, D), lambda i, m: (0, 0, 0)),
        scratch_shapes=[
            pltpu.VMEM((KV_SHARD, SKV, DH), jnp.float32),
            pltpu.VMEM((KV_SHARD, SKV, DH), jnp.float32),
            pltpu.VMEM((SQ, D), jnp.float32),
            pltpu.VMEM((N_DEV, SQ, QC), jnp.float32),
            pltpu.VMEM((2, SQ, QC), jnp.float32),
            pltpu.VMEM((1, SQ, QC), jnp.float32),
            pltpu.SemaphoreType.DMA((4,)),
            pltpu.SemaphoreType.DMA((N_DEV,)),
            pltpu.SemaphoreType.DMA((N_DEV,)),
        ],
    )
    return pl.pallas_call(
        body,
        grid_spec=grid_spec,
        out_shape=jax.ShapeDtypeStruct((1, SQ, D), jnp.float32),
        compiler_params=pltpu.CompilerParams(collective_id=0),
    )(my_pos.reshape(1), x, Wq, Wo, K_ext, V_ext)
